# baseline (speedup 1.0000x reference)
import jax
import jax.numpy as jnp
from jax import lax
from jax.experimental import pallas as pl

_GRID_SCALE = 0.0001
_BLOCK = 8192

_DN = (((0,), (0,)), ((), ()))


def _mlp_block(v_ref, w0, w1, w2, w3, w4, w5, w6, w7, w8, out_ref):
    p = v_ref[...].T
    s1 = jnp.sin(p)
    c1 = jnp.cos(p)
    s2 = 2.0 * s1 * c1
    c2 = 1.0 - 2.0 * s1 * s1
    s4 = 2.0 * s2 * c2
    c4 = 1.0 - 2.0 * s2 * s2
    s8 = 2.0 * s4 * c4
    c8 = 1.0 - 2.0 * s4 * s4
    s16 = 2.0 * s8 * c8
    c16 = 1.0 - 2.0 * s8 * s8
    e = jnp.concatenate(
        [p, s1, c1, s2, c2, s4, c4, s8, c8, s16, c16],
        axis=0).astype(jnp.bfloat16)
    h = jnp.maximum(
        lax.dot_general(w0[...].astype(jnp.bfloat16), e, _DN,
                        preferred_element_type=jnp.float32),
        0.0).astype(jnp.bfloat16)
    for w in (w1, w2, w3, w4, w5, w6, w7):
        h = jnp.maximum(
            lax.dot_general(w[...].astype(jnp.bfloat16), h, _DN,
                            preferred_element_type=jnp.float32),
            0.0).astype(jnp.bfloat16)
    out = lax.dot_general(w8[...].astype(jnp.bfloat16), h, _DN,
                          preferred_element_type=jnp.float32)
    v_def = p + _GRID_SCALE * jnp.tanh(out[1:4, :])
    out_ref[...] = jnp.concatenate([out[0:1, :], v_def], axis=0).T


def kernel(vertices, indices, W0, W1, W2, W3, W4, W5, W6, W7, W8):
    del indices
    n = vertices.shape[0]
    grid = (n + _BLOCK - 1) // _BLOCK
    ws = (W0, W1, W2, W3, W4, W5, W6, W7, W8)

    def w_spec(w):
        return pl.BlockSpec(w.shape, lambda i: (0, 0))

    return pl.pallas_call(
        _mlp_block,
        grid=(grid,),
        in_specs=[pl.BlockSpec((_BLOCK, 3), lambda i: (i, 0))]
        + [w_spec(w) for w in ws],
        out_specs=pl.BlockSpec((_BLOCK, 4), lambda i: (i, 0)),
        out_shape=jax.ShapeDtypeStruct((n, 4), jnp.float32),
    )(vertices, *ws)


# speedup vs baseline: 2.0709x; 2.0709x over previous
import jax
import jax.numpy as jnp
from jax import lax
from jax.experimental import pallas as pl

_GRID_SCALE = 0.0001
_BLOCK = 8192

_DN = (((0,), (0,)), ((), ()))


def _mlp_block(vt_ref, w0, w1, w2, w3, w4, w5, w6, w7, w8, out_ref):
    p = vt_ref[...]
    s1 = jnp.sin(p)
    c1 = jnp.cos(p)
    s2 = 2.0 * s1 * c1
    c2 = 1.0 - 2.0 * s1 * s1
    s4 = 2.0 * s2 * c2
    c4 = 1.0 - 2.0 * s2 * s2
    s8 = 2.0 * s4 * c4
    c8 = 1.0 - 2.0 * s4 * s4
    s16 = 2.0 * s8 * c8
    c16 = 1.0 - 2.0 * s8 * s8
    e = jnp.concatenate(
        [p, s1, c1, s2, c2, s4, c4, s8, c8, s16, c16],
        axis=0).astype(jnp.bfloat16)
    zero = jnp.bfloat16(0.0)
    h = jnp.maximum(
        lax.dot_general(w0[...].astype(jnp.bfloat16), e, _DN,
                        preferred_element_type=jnp.float32
                        ).astype(jnp.bfloat16), zero)
    for w in (w1, w2, w3, w4, w5, w6, w7):
        h = jnp.maximum(
            lax.dot_general(w[...].astype(jnp.bfloat16), h, _DN,
                            preferred_element_type=jnp.float32
                            ).astype(jnp.bfloat16), zero)
    out = lax.dot_general(w8[...].astype(jnp.bfloat16), h, _DN,
                          preferred_element_type=jnp.float32)
    v_def = p + _GRID_SCALE * jnp.tanh(out[1:4, :])
    out_ref[...] = jnp.concatenate([out[0:1, :], v_def], axis=0)


def kernel(vertices, indices, W0, W1, W2, W3, W4, W5, W6, W7, W8):
    del indices
    n = vertices.shape[0]
    grid = (n + _BLOCK - 1) // _BLOCK
    vt = vertices.T
    ws = (W0, W1, W2, W3, W4, W5, W6, W7, W8)

    def w_spec(w):
        return pl.BlockSpec(w.shape, lambda i: (0, 0))

    out_t = pl.pallas_call(
        _mlp_block,
        grid=(grid,),
        in_specs=[pl.BlockSpec((3, _BLOCK), lambda i: (0, i))]
        + [w_spec(w) for w in ws],
        out_specs=pl.BlockSpec((4, _BLOCK), lambda i: (0, i)),
        out_shape=jax.ShapeDtypeStruct((4, n), jnp.float32),
    )(vt, *ws)
    return out_t.T
